# baseline (speedup 1.0000x reference)
import functools

import jax
import jax.numpy as jnp
from jax import lax
from jax.experimental import pallas as pl
from jax.experimental.pallas import tpu as pltpu
from jax.experimental.pallas import tpu_sc as plsc

N = 10000
E = 320000
D = 128
HID = 128
H = 8
C = 16
NCLS = 2

SC_CORES = 2
SC_TILES = 16
NW = SC_CORES * SC_TILES
EPT = E // NW
B = 80
NCHUNK = EPT // B
NG = B // 16
NPA = (N // SC_TILES) // 8 * 8

_mesh = plsc.VectorSubcoreMesh(core_axis_name="c", subcore_axis_name="s")
_params = pltpu.CompilerParams(needs_layout_passes=False)


@functools.partial(
    pl.kernel,
    out_type=(
        jax.ShapeDtypeStruct((E * H,), jnp.float32),
        jax.ShapeDtypeStruct((NW * N * H,), jnp.float32),
    ),
    mesh=_mesh,
    compiler_params=_params,
    scratch_types=[
        pltpu.VMEM((B,), jnp.int32),
        pltpu.VMEM((B,), jnp.int32),
        pltpu.VMEM((B,), jnp.int32),
        pltpu.VMEM((B,), jnp.int32),
        pltpu.VMEM((2 * B,), jnp.int32),
        pltpu.VMEM((2 * B,), jnp.int32),
        pltpu.VMEM((2 * B, HID), jnp.float32),
        pltpu.VMEM((2 * B, HID), jnp.float32),
        pltpu.VMEM((H * B,), jnp.float32),
        pltpu.VMEM((H * B,), jnp.float32),
        pltpu.VMEM((N * H,), jnp.float32),
        pltpu.VMEM((B,), jnp.int32),
    ] + [pltpu.SemaphoreType.DMA] * 10,
)
def _sc_pass1(qk_hbm, dst_hbm, src_hbm, ex_hbm, den_hbm,
              dstA, dstB, srcA, srcB, icA, icB, qkA, qkB, exbA, exbB,
              den_tab, dstc,
              sDA, sDB, sSA, sSB, sQA, sQB, sKA, sKB, sEA, sEB):
    wid = lax.axis_index("s") * SC_CORES + lax.axis_index("c")
    base_e = wid * EPT
    zz = jnp.zeros((16,), jnp.float32)
    iota = lax.iota(jnp.int32, 16)

    dstR = (dstA, dstB)
    srcR = (srcA, srcB)
    icR = (icA, icB)
    qkR = (qkA, qkB)
    exbR = (exbA, exbB)
    sD = (sDA, sDB)
    sS = (sSA, sSB)
    sQ = (sQA, sQB)
    sE = (sEA, sEB)

    def zero_body(i, carry):
        den_tab[pl.ds(pl.multiple_of(i * 16, 16), 16)] = zz
        return carry

    lax.fori_loop(0, (N * H) // 16, zero_body, 0)

    def issue_idx(ci, b):
        eoff = pl.multiple_of(base_e + ci * B, 8)
        pltpu.async_copy(dst_hbm.at[pl.ds(eoff, B)], dstR[b], sD[b])
        pltpu.async_copy(src_hbm.at[pl.ds(eoff, B)], srcR[b], sS[b])

    def wait_idx(b):
        pltpu.make_async_copy(dst_hbm.at[pl.ds(0, B)], dstR[b], sD[b]).wait()
        pltpu.make_async_copy(src_hbm.at[pl.ds(0, B)], srcR[b], sS[b]).wait()

    def issue_gather(b):
        for j in range(NG):
            icR[b][pl.ds(j * 16, 16)] = dstR[b][pl.ds(j * 16, 16)]
            icR[b][pl.ds(B + j * 16, 16)] = srcR[b][pl.ds(j * 16, 16)] + N
        pltpu.async_copy(qk_hbm.at[icR[b]], qkR[b], sQ[b])

    def wait_gather(b):
        pltpu.make_async_copy(qk_hbm.at[icR[b]], qkR[b], sQ[b]).wait()

    def ex_slice(ci):
        exoff = pl.multiple_of((base_e * H) + ci * (H * B), 128)
        return ex_hbm.at[pl.ds(exoff, H * B)]

    def wait_ex(b):
        pltpu.make_async_copy(exbR[b], ex_hbm.at[pl.ds(0, H * B)],
                              sE[b]).wait()

    issue_idx(0, 0)
    wait_idx(0)
    issue_gather(0)
    issue_idx(1, 1)

    def pair(cio, carry):
        for b in (0, 1):
            ci = cio * 2 + b

            @pl.when(ci < NCHUNK)
            def _step():
                nb = 1 - b

                @pl.when(ci + 1 < NCHUNK)
                def _prefetch():
                    wait_idx(nb)
                    issue_gather(nb)

                wait_gather(b)
                for g in range(NG):
                    dstc[pl.ds(g * 16, 16)] = dstR[b][pl.ds(g * 16, 16)]

                @pl.when(ci + 2 < NCHUNK)
                def _next_idx():
                    issue_idx(ci + 2, b)

                @pl.when(ci >= 2)
                def _drain_ex():
                    wait_ex(b)

                qkr = qkR[b]
                exb = exbR[b]
                l2 = jnp.right_shift(iota, 1)
                hdv = (jnp.left_shift(jnp.bitwise_and(l2, 1), 2)
                       | jnp.bitwise_and(l2, 2)
                       | jnp.bitwise_and(jnp.right_shift(l2, 2), 1))
                lt8 = iota < 8
                m4 = jnp.bitwise_and(iota, 4) == 0
                m2 = jnp.bitwise_and(iota, 2) == 0
                m1 = jnp.bitwise_and(iota, 1) == 0

                def _lperm(x, s):
                    return x.at[jnp.bitwise_xor(iota, s)].get(
                        mode="promise_in_bounds")

                def _edge_reduce(e):
                    u = []
                    for h in range(H):
                        p = (qkr[e, pl.ds(h * C, C)] *
                             qkr[B + e, pl.ds(h * C, C)])
                        u.append(p + _lperm(p, 8))
                    v = [jnp.where(lt8, u[2 * i], u[2 * i + 1])
                         for i in range(4)]
                    w = [x + _lperm(x, 4) for x in v]
                    z = [jnp.where(m4, w[0], w[1]),
                         jnp.where(m4, w[2], w[3])]
                    y = [x + _lperm(x, 2) for x in z]
                    t = jnp.where(m2, y[0], y[1])
                    return t + _lperm(t, 1)

                def gbody(g, carry):
                    go = pl.multiple_of(g * 16, 16)
                    gf = pl.multiple_of(g * 128, 128)
                    dstv = dstc[pl.ds(go, 16)]
                    for ep in range(8):
                        f0 = _edge_reduce(go + 2 * ep)
                        f1 = _edge_reduce(go + 2 * ep + 1)
                        rc = jnp.where(m1, f0, f1)
                        exv = jnp.exp(rc * 0.25)
                        exb[pl.ds(gf + ep * 16, 16)] = exv
                        d0 = dstv.at[jnp.full((16,), 2 * ep, jnp.int32)].get(
                            mode="promise_in_bounds")
                        d1 = dstv.at[jnp.full((16,), 2 * ep + 1,
                                              jnp.int32)].get(
                            mode="promise_in_bounds")
                        ddp = jnp.where(m1, d0, d1)
                        plsc.addupdate_scatter(den_tab, [ddp * H + hdv], exv)
                    return carry

                lax.fori_loop(0, NG, gbody, 0)
                pltpu.async_copy(exb, ex_slice(ci), sE[b])
        return carry

    lax.fori_loop(0, (NCHUNK + 1) // 2, pair, 0)
    wait_ex(1)
    wait_ex(0)
    pltpu.sync_copy(den_tab,
                    den_hbm.at[pl.ds(pl.multiple_of(wid * (N * H), 128),
                                     N * H)])


@functools.partial(
    pl.kernel,
    out_type=jax.ShapeDtypeStruct((SC_CORES, N, HID), jnp.float32),
    mesh=_mesh,
    compiler_params=_params,
    scratch_types=[
        pltpu.VMEM((B,), jnp.int32),
        pltpu.VMEM((B,), jnp.int32),
        pltpu.VMEM((B,), jnp.int32),
        pltpu.VMEM((B,), jnp.int32),
        pltpu.VMEM((B, HID), jnp.float32),
        pltpu.VMEM((B, HID), jnp.float32),
        pltpu.VMEM((B, HID), jnp.float32),
        pltpu.VMEM((B, HID), jnp.float32),
        pltpu.VMEM((H * B,), jnp.float32),
        pltpu.VMEM((H * B,), jnp.float32),
        pltpu.VMEM_SHARED((N, HID), jnp.float32),
    ] + [pltpu.SemaphoreType.DMA] * 10,
)
def _sc_pass2(v_hbm, dst_hbm, src_hbm, ex_hbm, out_hbm,
              dstA, dstB, srcA, srcB, vrA, vrB, rbA, rbB, exbA, exbB,
              acc_sh,
              sDA, sDB, sSA, sSB, sVA, sVB, sXA, sXB, sAA, sAB):
    cid = lax.axis_index("c")
    sid = lax.axis_index("s")
    wid = sid * SC_CORES + cid
    base_e = wid * EPT
    zz = jnp.zeros((16,), jnp.float32)
    iota = lax.iota(jnp.int32, 16)

    dstR = (dstA, dstB)
    srcR = (srcA, srcB)
    vrR = (vrA, vrB)
    rbR = (rbA, rbB)
    exbR = (exbA, exbB)
    sD = (sDA, sDB)
    sS = (sSA, sSB)
    sV = (sVA, sVB)
    sX = (sXA, sXB)
    sA = (sAA, sAB)

    def zero_body(i, carry):
        r = i // (HID // 16)
        col = (i % (HID // 16)) * 16
        rbA[r, pl.ds(col, 16)] = zz
        return carry

    lax.fori_loop(0, B * (HID // 16), zero_body, 0)
    for z in range(7):
        pltpu.sync_copy(rbA, acc_sh.at[pl.ds(sid * NPA + z * B, B)])
    pltpu.sync_copy(rbA.at[pl.ds(0, NPA - 7 * B)],
                    acc_sh.at[pl.ds(sid * NPA + 7 * B, NPA - 7 * B)])

    @pl.when(sid == 0)
    def _zero_tail():
        pltpu.sync_copy(rbA.at[pl.ds(0, N - NPA * SC_TILES)],
                        acc_sh.at[pl.ds(NPA * SC_TILES, N - NPA * SC_TILES)])

    plsc.subcore_barrier()

    def issue_src(ci, b):
        eoff = pl.multiple_of(base_e + ci * B, 8)
        pltpu.async_copy(src_hbm.at[pl.ds(eoff, B)], srcR[b], sS[b])

    def wait_src(b):
        pltpu.make_async_copy(src_hbm.at[pl.ds(0, B)], srcR[b], sS[b]).wait()

    def issue_vex(ci, b):
        pltpu.async_copy(v_hbm.at[srcR[b]], vrR[b], sV[b])
        exoff = pl.multiple_of((base_e * H) + ci * (H * B), 128)
        pltpu.async_copy(ex_hbm.at[pl.ds(exoff, H * B)], exbR[b], sX[b])

    def wait_vex(b):
        pltpu.make_async_copy(v_hbm.at[srcR[b]], vrR[b], sV[b]).wait()
        pltpu.make_async_copy(ex_hbm.at[pl.ds(0, H * B)], exbR[b],
                              sX[b]).wait()

    def wait_scatter(b):
        pltpu.make_async_copy(rbR[b], acc_sh.at[dstR[b]], sA[b]).wait()

    issue_src(0, 0)
    wait_src(0)
    issue_vex(0, 0)
    issue_src(1, 1)

    def pair(cio, carry):
        for b in (0, 1):
            ci = cio * 2 + b

            @pl.when(ci < NCHUNK)
            def _step():
                nb = 1 - b

                @pl.when(ci + 1 < NCHUNK)
                def _prefetch():
                    wait_src(nb)
                    issue_vex(ci + 1, nb)

                @pl.when(ci >= 2)
                def _drain_scatter():
                    wait_scatter(b)

                eoff = pl.multiple_of(base_e + ci * B, 8)
                pltpu.async_copy(dst_hbm.at[pl.ds(eoff, B)], dstR[b], sD[b])

                wait_vex(b)

                @pl.when(ci + 2 < NCHUNK)
                def _next_src():
                    issue_src(ci + 2, b)

                vr = vrR[b]
                rb = rbR[b]
                exb = exbR[b]

                LANE0 = (0, 8, 4, 12, 2, 10, 6, 14)

                def gbody(g, carry):
                    go = pl.multiple_of(g * 16, 16)
                    gf = pl.multiple_of(g * 128, 128)
                    for ep in range(8):
                        pe = exb[pl.ds(gf + ep * 16, 16)]
                        for par in (0, 1):
                            e = go + 2 * ep + par
                            for h in range(H):
                                w = pe.at[jnp.full((16,), LANE0[h] + par,
                                                   jnp.int32)].get(
                                    mode="promise_in_bounds")
                                rb[e, pl.ds(h * C, C)] = (
                                    vr[e, pl.ds(h * C, C)] * w)
                    return carry

                lax.fori_loop(0, NG, gbody, 0)
                pltpu.make_async_copy(dst_hbm.at[pl.ds(0, B)], dstR[b],
                                      sD[b]).wait()
                pltpu.async_copy(rb, acc_sh.at[dstR[b]], sA[b], add=True)
        return carry

    lax.fori_loop(0, (NCHUNK + 1) // 2, pair, 0)
    wait_scatter(1)
    wait_scatter(0)
    plsc.subcore_barrier()
    pltpu.sync_copy(acc_sh.at[pl.ds(sid * NPA, NPA)],
                    out_hbm.at[cid, pl.ds(sid * NPA, NPA)])

    @pl.when(sid == 0)
    def _drain_tail():
        pltpu.sync_copy(acc_sh.at[pl.ds(NPA * SC_TILES, N - NPA * SC_TILES)],
                        out_hbm.at[cid, pl.ds(NPA * SC_TILES,
                                              N - NPA * SC_TILES)])


RB = 2000


def _dot(a, b):
    return jax.lax.dot_general(a, b, (((1,), (0,)), ((), ())),
                               preferred_element_type=jnp.float32)


def _tc_pre(x, W_in, b_in, Wq, bq, Wk, bk, Wv, bv, Ws, bs):
    def body(x_r, Wi, bi, Wq_, bq_, Wk_, bk_, Wv_, bv_, Ws_, bs_,
             h_r, q_r, k_r, v_r, s_r):
        h = _dot(x_r[...], Wi[...]) + bi[...]
        h_r[...] = h
        q_r[...] = _dot(h, Wq_[...]) + bq_[...]
        k_r[...] = _dot(h, Wk_[...]) + bk_[...]
        v_r[...] = _dot(h, Wv_[...]) + bv_[...]
        s_r[...] = _dot(h, Ws_[...]) + bs_[...]

    wspec = pl.BlockSpec((D, HID), lambda i: (0, 0))
    bspec = pl.BlockSpec((1, HID), lambda i: (0, 0))
    rspec = pl.BlockSpec((RB, HID), lambda i: (i, 0))
    return pl.pallas_call(
        body,
        grid=(N // RB,),
        in_specs=[pl.BlockSpec((RB, D), lambda i: (i, 0)),
                  wspec, bspec, wspec, bspec, wspec, bspec,
                  wspec, bspec, wspec, bspec],
        out_specs=[rspec] * 5,
        out_shape=[jax.ShapeDtypeStruct((N, HID), jnp.float32)] * 5,
    )(x, W_in, b_in, Wq, bq, Wk, bk, Wv, bv, Ws, bs)


def _tc_rden(den_p):
    def body(d_r, r_r):
        r_r[...] = 1.0 / (jnp.sum(d_r[...], axis=0, keepdims=True) + 1e-16)

    return pl.pallas_call(
        body,
        out_shape=jax.ShapeDtypeStruct((1, N * H), jnp.float32),
    )(den_p)


def _ln_block(y, g, be):
    mu = jnp.mean(y, axis=-1, keepdims=True)
    yc = y - mu
    var = jnp.mean(yc * yc, axis=-1, keepdims=True)
    return yc * jax.lax.rsqrt(var + 1e-5) * g + be


def _head_expand():
    row = lax.broadcasted_iota(jnp.int32, (H, HID), 0)
    lane = lax.broadcasted_iota(jnp.int32, (H, HID), 1)
    return (row == lane // C).astype(jnp.float32)


def _tc_mid(h, o0, o1, rd, sk, g, be, Wq, bq, Wk, bk, Wv, bv, Ws, bs):
    def body(h_r, o0_r, o1_r, rd_r, sk_r, g_, be_, Wq_, bq_, Wk_, bk_,
             Wv_, bv_, Ws_, bs_, hn_r, q_r, k_r, v_r, s_r):
        rdw = _dot(rd_r[...], _head_expand())
        y = h_r[...] + (o0_r[...] + o1_r[...]) * rdw + sk_r[...]
        hn = _ln_block(y, g_[...], be_[...])
        hn_r[...] = hn
        q_r[...] = _dot(hn, Wq_[...]) + bq_[...]
        k_r[...] = _dot(hn, Wk_[...]) + bk_[...]
        v_r[...] = _dot(hn, Wv_[...]) + bv_[...]
        s_r[...] = _dot(hn, Ws_[...]) + bs_[...]

    wspec = pl.BlockSpec((D, HID), lambda i: (0, 0))
    bspec = pl.BlockSpec((1, HID), lambda i: (0, 0))
    rspec = pl.BlockSpec((RB, HID), lambda i: (i, 0))
    dspec = pl.BlockSpec((RB, H), lambda i: (i, 0))
    return pl.pallas_call(
        body,
        grid=(N // RB,),
        in_specs=[rspec, rspec, rspec, dspec, rspec, bspec, bspec,
                  wspec, bspec, wspec, bspec, wspec, bspec, wspec, bspec],
        out_specs=[rspec] * 5,
        out_shape=[jax.ShapeDtypeStruct((N, HID), jnp.float32)] * 5,
    )(h, o0, o1, rd, sk, g, be, Wq, bq, Wk, bk, Wv, bv, Ws, bs)


def _tc_final(h, o0, o1, rd, sk, g, be, Wc1, bc1, Wc2, bc2):
    def body(h_r, o0_r, o1_r, rd_r, sk_r, g_, be_, W1, b1, W2, b2, out_r):
        rdw = _dot(rd_r[...], _head_expand())
        y = h_r[...] + (o0_r[...] + o1_r[...]) * rdw + sk_r[...]
        hn = _ln_block(y, g_[...], be_[...])
        z = jnp.maximum(_dot(hn, W1[...]) + b1[...], 0.0)
        out_r[...] = _dot(z, W2[...]) + b2[...]

    rspec = pl.BlockSpec((RB, HID), lambda i: (i, 0))
    bspec = pl.BlockSpec((1, HID), lambda i: (0, 0))
    dspec = pl.BlockSpec((RB, H), lambda i: (i, 0))
    return pl.pallas_call(
        body,
        grid=(N // RB,),
        in_specs=[rspec, rspec, rspec, dspec, rspec, bspec, bspec,
                  pl.BlockSpec((HID, HID), lambda i: (0, 0)), bspec,
                  pl.BlockSpec((HID, HID), lambda i: (0, 0)), bspec],
        out_specs=rspec,
        out_shape=jax.ShapeDtypeStruct((N, HID), jnp.float32),
    )(h, o0, o1, rd, sk, g, be, Wc1, bc1, Wc2, bc2)


def kernel(x, edge_index, W_in, b_in,
           Wq1, Wk1, Wv1, Ws1, bq1, bk1, bv1, bs1, g1, be1,
           Wq2, Wk2, Wv2, Ws2, bq2, bk2, bv2, bs2, g2, be2,
           Wc1, bc1, Wc2, bc2):
    src = edge_index[0]
    dst = edge_index[1]
    r2 = lambda v: v.reshape(1, -1)

    h, q1, k1, v1, sk1 = _tc_pre(x, W_in, r2(b_in), Wq1, r2(bq1),
                                 Wk1, r2(bk1), Wv1, r2(bv1), Ws1, r2(bs1))
    ex1, den1 = _sc_pass1(jnp.concatenate([q1, k1], axis=0), dst, src)
    rden1 = _tc_rden(den1.reshape(NW, N * H)).reshape(N, H)
    outp1 = _sc_pass2(v1, dst, src, ex1)

    hn, q2, k2, v2, sk2 = _tc_mid(h, outp1[0], outp1[1], rden1, sk1,
                                  r2(g1), r2(be1),
                                  Wq2, r2(bq2), Wk2, r2(bk2), Wv2, r2(bv2),
                                  Ws2, r2(bs2))
    ex2, den2 = _sc_pass1(jnp.concatenate([q2, k2], axis=0), dst, src)
    rden2 = _tc_rden(den2.reshape(NW, N * H)).reshape(N, H)
    outp2 = _sc_pass2(v2, dst, src, ex2)

    W1p = jnp.pad(Wc1, ((0, 0), (0, HID - Wc1.shape[1])))
    b1p = jnp.pad(r2(bc1), ((0, 0), (0, HID - bc1.shape[0])))
    W2p = jnp.pad(Wc2, ((0, HID - Wc2.shape[0]), (0, HID - Wc2.shape[1])))
    b2p = jnp.pad(r2(bc2), ((0, 0), (0, HID - bc2.shape[0])))
    logits = _tc_final(hn, outp2[0], outp2[1], rden2, sk2, r2(g2), r2(be2),
                       W1p, b1p, W2p, b2p)
    return logits[:, :NCLS]


# speedup vs baseline: 1.0266x; 1.0266x over previous
import functools

import jax
import jax.numpy as jnp
from jax import lax
from jax.experimental import pallas as pl
from jax.experimental.pallas import tpu as pltpu
from jax.experimental.pallas import tpu_sc as plsc

N = 10000
E = 320000
D = 128
HID = 128
H = 8
C = 16
NCLS = 2

SC_CORES = 2
SC_TILES = 16
NW = SC_CORES * SC_TILES
EPT = E // NW
B = 80
NCHUNK = EPT // B
NG = B // 16
NPA = (N // SC_TILES) // 8 * 8

_mesh = plsc.VectorSubcoreMesh(core_axis_name="c", subcore_axis_name="s")
_params = pltpu.CompilerParams(needs_layout_passes=False)


@functools.partial(
    pl.kernel,
    out_type=(
        jax.ShapeDtypeStruct((E * H,), jnp.float32),
        jax.ShapeDtypeStruct((NW * N * H,), jnp.float32),
    ),
    mesh=_mesh,
    compiler_params=_params,
    scratch_types=[
        pltpu.VMEM((B,), jnp.int32),
        pltpu.VMEM((B,), jnp.int32),
        pltpu.VMEM((B,), jnp.int32),
        pltpu.VMEM((B,), jnp.int32),
        pltpu.VMEM((B, HID), jnp.float32),
        pltpu.VMEM((B, HID), jnp.float32),
        pltpu.VMEM((B, HID), jnp.float32),
        pltpu.VMEM((B, HID), jnp.float32),
        pltpu.VMEM((H * B,), jnp.float32),
        pltpu.VMEM((H * B,), jnp.float32),
        pltpu.VMEM((N * H,), jnp.float32),
        pltpu.VMEM((B,), jnp.int32),
    ] + [pltpu.SemaphoreType.DMA] * 10,
)
def _sc_pass1(q_hbm, k_hbm, dst_hbm, src_hbm, ex_hbm, den_hbm,
              dstA, dstB, srcA, srcB, qrA, qrB, krA, krB, exbA, exbB,
              den_tab, dstc,
              sDA, sDB, sSA, sSB, sQA, sQB, sKA, sKB, sEA, sEB):
    wid = lax.axis_index("s") * SC_CORES + lax.axis_index("c")
    base_e = wid * EPT
    zz = jnp.zeros((16,), jnp.float32)
    iota = lax.iota(jnp.int32, 16)

    dstR = (dstA, dstB)
    srcR = (srcA, srcB)
    qrR = (qrA, qrB)
    krR = (krA, krB)
    exbR = (exbA, exbB)
    sD = (sDA, sDB)
    sS = (sSA, sSB)
    sQ = (sQA, sQB)
    sK = (sKA, sKB)
    sE = (sEA, sEB)

    def zero_body(i, carry):
        den_tab[pl.ds(pl.multiple_of(i * 16, 16), 16)] = zz
        return carry

    lax.fori_loop(0, (N * H) // 16, zero_body, 0)

    def issue_idx(ci, b):
        eoff = pl.multiple_of(base_e + ci * B, 8)
        pltpu.async_copy(dst_hbm.at[pl.ds(eoff, B)], dstR[b], sD[b])
        pltpu.async_copy(src_hbm.at[pl.ds(eoff, B)], srcR[b], sS[b])

    def wait_idx(b):
        pltpu.make_async_copy(dst_hbm.at[pl.ds(0, B)], dstR[b], sD[b]).wait()
        pltpu.make_async_copy(src_hbm.at[pl.ds(0, B)], srcR[b], sS[b]).wait()

    def issue_gather(b):
        pltpu.async_copy(q_hbm.at[dstR[b]], qrR[b], sQ[b])
        pltpu.async_copy(k_hbm.at[srcR[b]], krR[b], sK[b])

    def wait_gather(b):
        pltpu.make_async_copy(q_hbm.at[dstR[b]], qrR[b], sQ[b]).wait()
        pltpu.make_async_copy(k_hbm.at[srcR[b]], krR[b], sK[b]).wait()

    def ex_slice(ci):
        exoff = pl.multiple_of((base_e * H) + ci * (H * B), 128)
        return ex_hbm.at[pl.ds(exoff, H * B)]

    def wait_ex(b):
        pltpu.make_async_copy(exbR[b], ex_hbm.at[pl.ds(0, H * B)],
                              sE[b]).wait()

    issue_idx(0, 0)
    wait_idx(0)
    issue_gather(0)
    issue_idx(1, 1)

    def pair(cio, carry):
        for b in (0, 1):
            ci = cio * 2 + b

            @pl.when(ci < NCHUNK)
            def _step():
                nb = 1 - b

                @pl.when(ci + 1 < NCHUNK)
                def _prefetch():
                    wait_idx(nb)
                    issue_gather(nb)

                wait_gather(b)
                for g in range(NG):
                    dstc[pl.ds(g * 16, 16)] = dstR[b][pl.ds(g * 16, 16)]

                @pl.when(ci + 2 < NCHUNK)
                def _next_idx():
                    issue_idx(ci + 2, b)

                @pl.when(ci >= 2)
                def _drain_ex():
                    wait_ex(b)

                qr = qrR[b]
                kr = krR[b]
                exb = exbR[b]
                l2 = jnp.right_shift(iota, 1)
                hdv = (jnp.left_shift(jnp.bitwise_and(l2, 1), 2)
                       | jnp.bitwise_and(l2, 2)
                       | jnp.bitwise_and(jnp.right_shift(l2, 2), 1))
                lt8 = iota < 8
                m4 = jnp.bitwise_and(iota, 4) == 0
                m2 = jnp.bitwise_and(iota, 2) == 0
                m1 = jnp.bitwise_and(iota, 1) == 0

                def _lperm(x, s):
                    return x.at[jnp.bitwise_xor(iota, s)].get(
                        mode="promise_in_bounds")

                def _edge_reduce(e):
                    u = []
                    for h in range(H):
                        p = qr[e, pl.ds(h * C, C)] * kr[e, pl.ds(h * C, C)]
                        u.append(p + _lperm(p, 8))
                    v = [jnp.where(lt8, u[2 * i], u[2 * i + 1])
                         for i in range(4)]
                    w = [x + _lperm(x, 4) for x in v]
                    z = [jnp.where(m4, w[0], w[1]),
                         jnp.where(m4, w[2], w[3])]
                    y = [x + _lperm(x, 2) for x in z]
                    t = jnp.where(m2, y[0], y[1])
                    return t + _lperm(t, 1)

                def gbody(g, carry):
                    go = pl.multiple_of(g * 16, 16)
                    gf = pl.multiple_of(g * 128, 128)
                    dstv = dstc[pl.ds(go, 16)]
                    for ep in range(8):
                        f0 = _edge_reduce(go + 2 * ep)
                        f1 = _edge_reduce(go + 2 * ep + 1)
                        rc = jnp.where(m1, f0, f1)
                        exv = jnp.exp(rc * 0.25)
                        exb[pl.ds(gf + ep * 16, 16)] = exv
                        d0 = dstv.at[jnp.full((16,), 2 * ep, jnp.int32)].get(
                            mode="promise_in_bounds")
                        d1 = dstv.at[jnp.full((16,), 2 * ep + 1,
                                              jnp.int32)].get(
                            mode="promise_in_bounds")
                        ddp = jnp.where(m1, d0, d1)
                        plsc.addupdate_scatter(den_tab, [ddp * H + hdv], exv)
                    return carry

                lax.fori_loop(0, NG, gbody, 0)
                pltpu.async_copy(exb, ex_slice(ci), sE[b])
        return carry

    lax.fori_loop(0, (NCHUNK + 1) // 2, pair, 0)
    wait_ex(1)
    wait_ex(0)
    pltpu.sync_copy(den_tab,
                    den_hbm.at[pl.ds(pl.multiple_of(wid * (N * H), 128),
                                     N * H)])


@functools.partial(
    pl.kernel,
    out_type=jax.ShapeDtypeStruct((SC_CORES, N, HID), jnp.float32),
    mesh=_mesh,
    compiler_params=_params,
    scratch_types=[
        pltpu.VMEM((B,), jnp.int32),
        pltpu.VMEM((B,), jnp.int32),
        pltpu.VMEM((B,), jnp.int32),
        pltpu.VMEM((B,), jnp.int32),
        pltpu.VMEM((B, HID), jnp.float32),
        pltpu.VMEM((B, HID), jnp.float32),
        pltpu.VMEM((B, HID), jnp.float32),
        pltpu.VMEM((B, HID), jnp.float32),
        pltpu.VMEM((H * B,), jnp.float32),
        pltpu.VMEM((H * B,), jnp.float32),
        pltpu.VMEM_SHARED((N, HID), jnp.float32),
    ] + [pltpu.SemaphoreType.DMA] * 10,
)
def _sc_pass2(v_hbm, dst_hbm, src_hbm, ex_hbm, out_hbm,
              dstA, dstB, srcA, srcB, vrA, vrB, rbA, rbB, exbA, exbB,
              acc_sh,
              sDA, sDB, sSA, sSB, sVA, sVB, sXA, sXB, sAA, sAB):
    cid = lax.axis_index("c")
    sid = lax.axis_index("s")
    wid = sid * SC_CORES + cid
    base_e = wid * EPT
    zz = jnp.zeros((16,), jnp.float32)
    iota = lax.iota(jnp.int32, 16)

    dstR = (dstA, dstB)
    srcR = (srcA, srcB)
    vrR = (vrA, vrB)
    rbR = (rbA, rbB)
    exbR = (exbA, exbB)
    sD = (sDA, sDB)
    sS = (sSA, sSB)
    sV = (sVA, sVB)
    sX = (sXA, sXB)
    sA = (sAA, sAB)

    def zero_body(i, carry):
        r = i // (HID // 16)
        col = (i % (HID // 16)) * 16
        rbA[r, pl.ds(col, 16)] = zz
        return carry

    lax.fori_loop(0, B * (HID // 16), zero_body, 0)
    for z in range(7):
        pltpu.sync_copy(rbA, acc_sh.at[pl.ds(sid * NPA + z * B, B)])
    pltpu.sync_copy(rbA.at[pl.ds(0, NPA - 7 * B)],
                    acc_sh.at[pl.ds(sid * NPA + 7 * B, NPA - 7 * B)])

    @pl.when(sid == 0)
    def _zero_tail():
        pltpu.sync_copy(rbA.at[pl.ds(0, N - NPA * SC_TILES)],
                        acc_sh.at[pl.ds(NPA * SC_TILES, N - NPA * SC_TILES)])

    plsc.subcore_barrier()

    def issue_src(ci, b):
        eoff = pl.multiple_of(base_e + ci * B, 8)
        pltpu.async_copy(src_hbm.at[pl.ds(eoff, B)], srcR[b], sS[b])

    def wait_src(b):
        pltpu.make_async_copy(src_hbm.at[pl.ds(0, B)], srcR[b], sS[b]).wait()

    def issue_vex(ci, b):
        pltpu.async_copy(v_hbm.at[srcR[b]], vrR[b], sV[b])
        exoff = pl.multiple_of((base_e * H) + ci * (H * B), 128)
        pltpu.async_copy(ex_hbm.at[pl.ds(exoff, H * B)], exbR[b], sX[b])

    def wait_vex(b):
        pltpu.make_async_copy(v_hbm.at[srcR[b]], vrR[b], sV[b]).wait()
        pltpu.make_async_copy(ex_hbm.at[pl.ds(0, H * B)], exbR[b],
                              sX[b]).wait()

    def wait_scatter(b):
        pltpu.make_async_copy(rbR[b], acc_sh.at[dstR[b]], sA[b]).wait()

    issue_src(0, 0)
    wait_src(0)
    issue_vex(0, 0)
    issue_src(1, 1)

    def pair(cio, carry):
        for b in (0, 1):
            ci = cio * 2 + b

            @pl.when(ci < NCHUNK)
            def _step():
                nb = 1 - b

                @pl.when(ci + 1 < NCHUNK)
                def _prefetch():
                    wait_src(nb)
                    issue_vex(ci + 1, nb)

                @pl.when(ci >= 2)
                def _drain_scatter():
                    wait_scatter(b)

                eoff = pl.multiple_of(base_e + ci * B, 8)
                pltpu.async_copy(dst_hbm.at[pl.ds(eoff, B)], dstR[b], sD[b])

                wait_vex(b)

                @pl.when(ci + 2 < NCHUNK)
                def _next_src():
                    issue_src(ci + 2, b)

                vr = vrR[b]
                rb = rbR[b]
                exb = exbR[b]

                LANE0 = (0, 8, 4, 12, 2, 10, 6, 14)

                def gbody(g, carry):
                    go = pl.multiple_of(g * 16, 16)
                    gf = pl.multiple_of(g * 128, 128)
                    for ep in range(8):
                        pe = exb[pl.ds(gf + ep * 16, 16)]
                        for par in (0, 1):
                            e = go + 2 * ep + par
                            for h in range(H):
                                w = pe.at[jnp.full((16,), LANE0[h] + par,
                                                   jnp.int32)].get(
                                    mode="promise_in_bounds")
                                rb[e, pl.ds(h * C, C)] = (
                                    vr[e, pl.ds(h * C, C)] * w)
                    return carry

                lax.fori_loop(0, NG, gbody, 0)
                pltpu.make_async_copy(dst_hbm.at[pl.ds(0, B)], dstR[b],
                                      sD[b]).wait()
                pltpu.async_copy(rb, acc_sh.at[dstR[b]], sA[b], add=True)
        return carry

    lax.fori_loop(0, (NCHUNK + 1) // 2, pair, 0)
    wait_scatter(1)
    wait_scatter(0)
    plsc.subcore_barrier()
    pltpu.sync_copy(acc_sh.at[pl.ds(sid * NPA, NPA)],
                    out_hbm.at[cid, pl.ds(sid * NPA, NPA)])

    @pl.when(sid == 0)
    def _drain_tail():
        pltpu.sync_copy(acc_sh.at[pl.ds(NPA * SC_TILES, N - NPA * SC_TILES)],
                        out_hbm.at[cid, pl.ds(NPA * SC_TILES,
                                              N - NPA * SC_TILES)])


RB = 2000


def _dot(a, b):
    return jax.lax.dot_general(a, b, (((1,), (0,)), ((), ())),
                               preferred_element_type=jnp.float32)


def _tc_pre(x, W_in, b_in, Wq, bq, Wk, bk, Wv, bv, Ws, bs):
    def body(x_r, Wi, bi, Wq_, bq_, Wk_, bk_, Wv_, bv_, Ws_, bs_,
             h_r, q_r, k_r, v_r, s_r):
        h = _dot(x_r[...], Wi[...]) + bi[...]
        h_r[...] = h
        q_r[...] = _dot(h, Wq_[...]) + bq_[...]
        k_r[...] = _dot(h, Wk_[...]) + bk_[...]
        v_r[...] = _dot(h, Wv_[...]) + bv_[...]
        s_r[...] = _dot(h, Ws_[...]) + bs_[...]

    wspec = pl.BlockSpec((D, HID), lambda i: (0, 0))
    bspec = pl.BlockSpec((1, HID), lambda i: (0, 0))
    rspec = pl.BlockSpec((RB, HID), lambda i: (i, 0))
    return pl.pallas_call(
        body,
        grid=(N // RB,),
        in_specs=[pl.BlockSpec((RB, D), lambda i: (i, 0)),
                  wspec, bspec, wspec, bspec, wspec, bspec,
                  wspec, bspec, wspec, bspec],
        out_specs=[rspec] * 5,
        out_shape=[jax.ShapeDtypeStruct((N, HID), jnp.float32)] * 5,
    )(x, W_in, b_in, Wq, bq, Wk, bk, Wv, bv, Ws, bs)


def _tc_rden(den_p):
    def body(d_r, r_r):
        r_r[...] = 1.0 / (jnp.sum(d_r[...], axis=0, keepdims=True) + 1e-16)

    return pl.pallas_call(
        body,
        out_shape=jax.ShapeDtypeStruct((1, N * H), jnp.float32),
    )(den_p)


def _ln_block(y, g, be):
    mu = jnp.mean(y, axis=-1, keepdims=True)
    yc = y - mu
    var = jnp.mean(yc * yc, axis=-1, keepdims=True)
    return yc * jax.lax.rsqrt(var + 1e-5) * g + be


def _head_expand():
    row = lax.broadcasted_iota(jnp.int32, (H, HID), 0)
    lane = lax.broadcasted_iota(jnp.int32, (H, HID), 1)
    return (row == lane // C).astype(jnp.float32)


def _tc_mid(h, o0, o1, rd, sk, g, be, Wq, bq, Wk, bk, Wv, bv, Ws, bs):
    def body(h_r, o0_r, o1_r, rd_r, sk_r, g_, be_, Wq_, bq_, Wk_, bk_,
             Wv_, bv_, Ws_, bs_, hn_r, q_r, k_r, v_r, s_r):
        rdw = _dot(rd_r[...], _head_expand())
        y = h_r[...] + (o0_r[...] + o1_r[...]) * rdw + sk_r[...]
        hn = _ln_block(y, g_[...], be_[...])
        hn_r[...] = hn
        q_r[...] = _dot(hn, Wq_[...]) + bq_[...]
        k_r[...] = _dot(hn, Wk_[...]) + bk_[...]
        v_r[...] = _dot(hn, Wv_[...]) + bv_[...]
        s_r[...] = _dot(hn, Ws_[...]) + bs_[...]

    wspec = pl.BlockSpec((D, HID), lambda i: (0, 0))
    bspec = pl.BlockSpec((1, HID), lambda i: (0, 0))
    rspec = pl.BlockSpec((RB, HID), lambda i: (i, 0))
    dspec = pl.BlockSpec((RB, H), lambda i: (i, 0))
    return pl.pallas_call(
        body,
        grid=(N // RB,),
        in_specs=[rspec, rspec, rspec, dspec, rspec, bspec, bspec,
                  wspec, bspec, wspec, bspec, wspec, bspec, wspec, bspec],
        out_specs=[rspec] * 5,
        out_shape=[jax.ShapeDtypeStruct((N, HID), jnp.float32)] * 5,
    )(h, o0, o1, rd, sk, g, be, Wq, bq, Wk, bk, Wv, bv, Ws, bs)


def _tc_final(h, o0, o1, rd, sk, g, be, Wc1, bc1, Wc2, bc2):
    def body(h_r, o0_r, o1_r, rd_r, sk_r, g_, be_, W1, b1, W2, b2, out_r):
        rdw = _dot(rd_r[...], _head_expand())
        y = h_r[...] + (o0_r[...] + o1_r[...]) * rdw + sk_r[...]
        hn = _ln_block(y, g_[...], be_[...])
        z = jnp.maximum(_dot(hn, W1[...]) + b1[...], 0.0)
        out_r[...] = _dot(z, W2[...]) + b2[...]

    rspec = pl.BlockSpec((RB, HID), lambda i: (i, 0))
    bspec = pl.BlockSpec((1, HID), lambda i: (0, 0))
    dspec = pl.BlockSpec((RB, H), lambda i: (i, 0))
    return pl.pallas_call(
        body,
        grid=(N // RB,),
        in_specs=[rspec, rspec, rspec, dspec, rspec, bspec, bspec,
                  pl.BlockSpec((HID, HID), lambda i: (0, 0)), bspec,
                  pl.BlockSpec((HID, HID), lambda i: (0, 0)), bspec],
        out_specs=rspec,
        out_shape=jax.ShapeDtypeStruct((N, HID), jnp.float32),
    )(h, o0, o1, rd, sk, g, be, Wc1, bc1, Wc2, bc2)


def kernel(x, edge_index, W_in, b_in,
           Wq1, Wk1, Wv1, Ws1, bq1, bk1, bv1, bs1, g1, be1,
           Wq2, Wk2, Wv2, Ws2, bq2, bk2, bv2, bs2, g2, be2,
           Wc1, bc1, Wc2, bc2):
    src = edge_index[0]
    dst = edge_index[1]
    r2 = lambda v: v.reshape(1, -1)

    h, q1, k1, v1, sk1 = _tc_pre(x, W_in, r2(b_in), Wq1, r2(bq1),
                                 Wk1, r2(bk1), Wv1, r2(bv1), Ws1, r2(bs1))
    ex1, den1 = _sc_pass1(q1, k1, dst, src)
    rden1 = _tc_rden(den1.reshape(NW, N * H)).reshape(N, H)
    outp1 = _sc_pass2(v1, dst, src, ex1)

    hn, q2, k2, v2, sk2 = _tc_mid(h, outp1[0], outp1[1], rden1, sk1,
                                  r2(g1), r2(be1),
                                  Wq2, r2(bq2), Wk2, r2(bk2), Wv2, r2(bv2),
                                  Ws2, r2(bs2))
    ex2, den2 = _sc_pass1(q2, k2, dst, src)
    rden2 = _tc_rden(den2.reshape(NW, N * H)).reshape(N, H)
    outp2 = _sc_pass2(v2, dst, src, ex2)

    W1p = jnp.pad(Wc1, ((0, 0), (0, HID - Wc1.shape[1])))
    b1p = jnp.pad(r2(bc1), ((0, 0), (0, HID - bc1.shape[0])))
    W2p = jnp.pad(Wc2, ((0, HID - Wc2.shape[0]), (0, HID - Wc2.shape[1])))
    b2p = jnp.pad(r2(bc2), ((0, 0), (0, HID - bc2.shape[0])))
    logits = _tc_final(hn, outp2[0], outp2[1], rden2, sk2, r2(g2), r2(be2),
                       W1p, b1p, W2p, b2p)
    return logits[:, :NCLS]
